# baseline (speedup 1.0000x reference)
import functools

import jax
import jax.numpy as jnp
from jax import lax
from jax.experimental import pallas as pl
from jax.experimental.pallas import tpu as pltpu
from jax.experimental.pallas import tpu_sc as plsc

N = 10000
E = 160000
D = 512
H = 2
C = 128
ETOT = E + N
NSC = 16
GPB = 4
BPB = GPB * 16
EPT = ((ETOT + NSC * 2 * BPB - 1) // (NSC * 2 * BPB)) * 2 * BPB
EPAD = EPT * NSC
NBAT = EPT // BPB
NP = 10240
ROWB = NP // NSC
ZR = 32
ZFW = 2048


def _proj_body(x_ref, w_ref, atts_ref, attd_ref, h_ref, as_ref, ad_ref):
    h = jnp.dot(x_ref[...], w_ref[...], preferred_element_type=jnp.float32)
    h0 = h[:, :C]
    h1 = h[:, C:]
    h_ref[0] = h0
    h_ref[1] = h1
    s0 = jnp.sum(h0 * atts_ref[0][None, :], axis=1, keepdims=True)
    s1 = jnp.sum(h1 * atts_ref[1][None, :], axis=1, keepdims=True)
    d0 = jnp.sum(h0 * attd_ref[0][None, :], axis=1, keepdims=True)
    d1 = jnp.sum(h1 * attd_ref[1][None, :], axis=1, keepdims=True)
    as_ref[...] = jnp.concatenate([s0, s1], axis=1)
    ad_ref[...] = jnp.concatenate([d0, d1], axis=1)


def _project(x, W, att_src, att_dst):
    B = 1000
    grid = (N // B,)
    return pl.pallas_call(
        _proj_body,
        grid=grid,
        in_specs=[
            pl.BlockSpec((B, D), lambda i: (i, 0)),
            pl.BlockSpec((D, H * C), lambda i: (0, 0)),
            pl.BlockSpec((H, C), lambda i: (0, 0)),
            pl.BlockSpec((H, C), lambda i: (0, 0)),
        ],
        out_specs=[
            pl.BlockSpec((H, B, C), lambda i: (0, i, 0)),
            pl.BlockSpec((B, H), lambda i: (i, 0)),
            pl.BlockSpec((B, H), lambda i: (i, 0)),
        ],
        out_shape=[
            jax.ShapeDtypeStruct((H, N, C), jnp.float32),
            jax.ShapeDtypeStruct((N, H), jnp.float32),
            jax.ShapeDtypeStruct((N, H), jnp.float32),
        ],
    )(x, W, att_src, att_dst)


def _edge_body(src_hbm, dst_hbm, asrc_hbm, adst_hbm, h_hbm, agg_hbm,
               src_v, dst_v, rows_v, asg_v, adg_v, sxp_v, den_v,
               coef_v, zflat_v, sem_r, sem_g, sem_d, sem_s, den_sh, out_sh):
    c = lax.axis_index("c")
    s = lax.axis_index("s")
    base = s * EPT
    iota = lax.iota(jnp.int32, 16)
    zf16 = jnp.zeros((16,), jnp.float32)

    pltpu.sync_copy(src_hbm.at[pl.ds(base, EPT)], src_v)
    pltpu.sync_copy(dst_hbm.at[pl.ds(base, EPT)], dst_v)

    def _zero_zflat(i, _):
        zflat_v[pl.ds(i * 16, 16)] = zf16
        return 0
    lax.fori_loop(0, ZFW // 16, _zero_zflat, 0)

    def _zero_rows(i, _):
        rows_v[i // 8, pl.ds((i % 8) * 16, 16)] = zf16
        return 0
    lax.fori_loop(0, 2 * BPB * 8, _zero_rows, 0)

    def _zero_out(i, _):
        pltpu.sync_copy(rows_v,
                        out_sh.at[pl.ds(s * ROWB + i * 2 * BPB, 2 * BPB)])
        return 0
    lax.fori_loop(0, ROWB // (2 * BPB), _zero_out, 0)

    @pl.when(s == 0)
    def _():
        def _zero_den(i, _):
            pltpu.sync_copy(zflat_v, den_sh.at[pl.ds(i * ZFW, ZFW)])
            return 0
        lax.fori_loop(0, NP // ZFW, _zero_den, 0)

    plsc.subcore_barrier()

    def _gather_logits(b, slot):
        ds_ = []
        for g in range(GPB):
            sv = src_v[pl.ds(b * BPB + g * 16, 16)]
            dv = dst_v[pl.ds(b * BPB + g * 16, 16)]
            ds_.append(pltpu.async_copy(
                asrc_hbm.at[sv * H + c],
                asg_v.at[pl.ds(slot * BPB + g * 16, 16)], sem_g))
            ds_.append(pltpu.async_copy(
                adst_hbm.at[dv * H + c],
                adg_v.at[pl.ds(slot * BPB + g * 16, 16)], sem_g))
        return ds_

    def _sxp_group(b, g, slot):
        off = slot * BPB + g * 16
        al = asg_v[pl.ds(off, 16)] + adg_v[pl.ds(off, 16)]
        al = jnp.where(al >= 0.0, al, 0.2 * al)
        sxp = jnp.exp(al)
        valid = (base + b * BPB + g * 16 + iota) < ETOT
        return jnp.where(valid, sxp, 0.0)

    def _half_p1(b, slot):
        for g in range(GPB):
            sxp_v[pl.ds(slot * BPB + g * 16, 16)] = _sxp_group(b, g, slot)
        sc_ = []
        for g in range(GPB):
            dv = dst_v[pl.ds(b * BPB + g * 16, 16)]
            sc_.append(pltpu.async_copy(
                sxp_v.at[pl.ds(slot * BPB + g * 16, 16)], den_sh.at[dv],
                sem_s, add=True))
        return sc_

    def _p1(i, _):
        ga = _gather_logits(2 * i, 0)
        gb = _gather_logits(2 * i + 1, 1)
        for d in ga:
            d.wait()
        sa = _half_p1(2 * i, 0)
        for d in gb:
            d.wait()
        sb = _half_p1(2 * i + 1, 1)
        for d in sa + sb:
            d.wait()
        return 0
    lax.fori_loop(0, NBAT // 2, _p1, 0)

    plsc.subcore_barrier()

    def _gather_msgs(b, slot):
        ds_ = [pltpu.async_copy(
                   h_hbm.at[c].at[src_v.at[pl.ds(b * BPB, BPB)]],
                   rows_v.at[pl.ds(slot * BPB, BPB), :], sem_r),
               pltpu.async_copy(
                   den_sh.at[dst_v.at[pl.ds(b * BPB, BPB)]],
                   den_v.at[pl.ds(slot * BPB, BPB)], sem_d)]
        return ds_ + _gather_logits(b, slot)

    def _half_p2(b, slot):
        for g in range(GPB):
            off = slot * BPB + g * 16
            sxp = _sxp_group(b, g, slot)
            coef_v[pl.ds(16, 16)] = sxp / (den_v[pl.ds(off, 16)] + 1e-16)
            for r in range(16):
                cv = plsc.load_gather(coef_v,
                                      [jnp.full((16,), 16 + r, jnp.int32)])
                for q in range(8):
                    rows_v[off + r, pl.ds(q * 16, 16)] = (
                        rows_v[off + r, pl.ds(q * 16, 16)] * cv)
        sc_ = []
        for g in range(GPB):
            dv = dst_v[pl.ds(b * BPB + g * 16, 16)]
            sc_.append(pltpu.async_copy(
                rows_v.at[pl.ds(slot * BPB + g * 16, 16), :], out_sh.at[dv],
                sem_s, add=True))
        return sc_

    def _p2(i, _):
        ga = _gather_msgs(2 * i, 0)
        gb = _gather_msgs(2 * i + 1, 1)
        for d in ga:
            d.wait()
        sa = _half_p2(2 * i, 0)
        for d in gb:
            d.wait()
        sb = _half_p2(2 * i + 1, 1)
        for d in sa + sb:
            d.wait()
        return 0
    lax.fori_loop(0, NBAT // 2, _p2, 0)

    plsc.subcore_barrier()
    pltpu.sync_copy(out_sh.at[pl.ds(s * ROWB, ROWB)],
                    agg_hbm.at[c, pl.ds(s * ROWB, ROWB), :])


def _edge_aggregate(src, dst, asrc_flat, adst_flat, h_flat):
    mesh = plsc.VectorSubcoreMesh(core_axis_name="c", subcore_axis_name="s")
    f = pl.kernel(
        _edge_body,
        out_type=jax.ShapeDtypeStruct((H, NP, C), jnp.float32),
        mesh=mesh,
        compiler_params=pltpu.CompilerParams(use_tc_tiling_on_sc=False,
                                             needs_layout_passes=False),
        scratch_types=[
            pltpu.VMEM((EPT,), jnp.int32),
            pltpu.VMEM((EPT,), jnp.int32),
            pltpu.VMEM((2 * BPB, C), jnp.float32),
            pltpu.VMEM((2 * BPB,), jnp.float32),
            pltpu.VMEM((2 * BPB,), jnp.float32),
            pltpu.VMEM((2 * BPB,), jnp.float32),
            pltpu.VMEM((2 * BPB,), jnp.float32),
            pltpu.VMEM((32,), jnp.float32),
            pltpu.VMEM((ZFW,), jnp.float32),
            pltpu.SemaphoreType.DMA,
            pltpu.SemaphoreType.DMA,
            pltpu.SemaphoreType.DMA,
            pltpu.SemaphoreType.DMA,
            pltpu.VMEM_SHARED((NP,), jnp.float32),
            pltpu.VMEM_SHARED((NP, C), jnp.float32),
        ],
    )
    return f(src, dst, asrc_flat, adst_flat, h_flat)


def _mlp_body(agg_ref, bias_ref, w1_ref, b1_ref, w2_ref, b2_ref, z_ref):
    feat = jnp.concatenate([agg_ref[0], agg_ref[1]], axis=1)
    out = feat + bias_ref[...]
    out = jnp.where(out >= 0.0, out, 0.01 * out)
    z1 = jnp.dot(out, w1_ref[...], preferred_element_type=jnp.float32) + b1_ref[...]
    z1 = jnp.where(z1 >= 0.0, z1, 0.01 * z1)
    z_ref[...] = jnp.dot(z1, w2_ref[...], preferred_element_type=jnp.float32) + b2_ref[...]


def _mlp(agg, bias, W1, b1, W2, b2):
    B = 1000
    return pl.pallas_call(
        _mlp_body,
        grid=(N // B,),
        in_specs=[
            pl.BlockSpec((H, B, C), lambda i: (0, i, 0)),
            pl.BlockSpec((1, H * C), lambda i: (0, 0)),
            pl.BlockSpec((H * C, 64), lambda i: (0, 0)),
            pl.BlockSpec((1, 64), lambda i: (0, 0)),
            pl.BlockSpec((64, 3), lambda i: (0, 0)),
            pl.BlockSpec((1, 3), lambda i: (0, 0)),
        ],
        out_specs=pl.BlockSpec((B, 3), lambda i: (i, 0)),
        out_shape=jax.ShapeDtypeStruct((N, 3), jnp.float32),
    )(agg, bias, W1, b1, W2, b2)


def _cdist_body(zi_ref, zjt_ref, d_ref):
    zi = zi_ref[...]
    zjt = zjt_ref[...]
    g = jnp.dot(zi, zjt, preferred_element_type=jnp.float32)
    sqi = jnp.sum(zi * zi, axis=1, keepdims=True)
    sqj = jnp.sum(zjt * zjt, axis=0, keepdims=True)
    d2 = sqi + sqj - 2.0 * g
    d2 = jnp.maximum(d2, 0.0)
    d = jnp.sqrt(jnp.where(d2 > 0.0, d2, 1.0))
    d_ref[...] = jnp.where(d2 > 0.0, d, 0.0)


def _cdist(z, zT):
    B = 1024
    g = (N + B - 1) // B
    return pl.pallas_call(
        _cdist_body,
        grid=(g, g),
        in_specs=[
            pl.BlockSpec((B, 3), lambda i, j: (i, 0)),
            pl.BlockSpec((3, B), lambda i, j: (0, j)),
        ],
        out_specs=pl.BlockSpec((B, B), lambda i, j: (i, j)),
        out_shape=jax.ShapeDtypeStruct((N, N), jnp.float32),
    )(z, zT)


def kernel(x, edge_index, W, att_src, att_dst, bias, W1, b1, W2, b2):
    loop = jnp.arange(N, dtype=edge_index.dtype)
    pad = jnp.zeros((EPAD - ETOT,), dtype=edge_index.dtype)
    src = jnp.concatenate([edge_index[0], loop, pad])
    dst = jnp.concatenate([edge_index[1], loop, pad])

    h, a_src, a_dst = _project(x, W, att_src.reshape(H, C), att_dst.reshape(H, C))
    agg = _edge_aggregate(src, dst, a_src.reshape(-1), a_dst.reshape(-1), h)
    z = _mlp(agg, bias.reshape(1, H * C), W1, b1.reshape(1, 64), W2,
             b2.reshape(1, 3))
    return _cdist(z, z.T)


# speedup vs baseline: 1.0085x; 1.0085x over previous
import functools

import jax
import jax.numpy as jnp
from jax import lax
from jax.experimental import pallas as pl
from jax.experimental.pallas import tpu as pltpu
from jax.experimental.pallas import tpu_sc as plsc

N = 10000
E = 160000
D = 512
H = 2
C = 128
ETOT = E + N
NSC = 16
GPB = 4
BPB = GPB * 16
EPT = ((ETOT + NSC * BPB - 1) // (NSC * BPB)) * BPB
EPAD = EPT * NSC
NBAT = EPT // BPB
NP = 10240
ROWB = NP // NSC
ZR = 32
ZFW = 2048


def _proj_body(x_ref, w_ref, atts_ref, attd_ref, h_ref, as_ref, ad_ref):
    h = jnp.dot(x_ref[...], w_ref[...], preferred_element_type=jnp.float32)
    h0 = h[:, :C]
    h1 = h[:, C:]
    h_ref[0] = h0
    h_ref[1] = h1
    s0 = jnp.sum(h0 * atts_ref[0][None, :], axis=1, keepdims=True)
    s1 = jnp.sum(h1 * atts_ref[1][None, :], axis=1, keepdims=True)
    d0 = jnp.sum(h0 * attd_ref[0][None, :], axis=1, keepdims=True)
    d1 = jnp.sum(h1 * attd_ref[1][None, :], axis=1, keepdims=True)
    as_ref[...] = jnp.concatenate([s0, s1], axis=1)
    ad_ref[...] = jnp.concatenate([d0, d1], axis=1)


def _project(x, W, att_src, att_dst):
    B = 1000
    grid = (N // B,)
    return pl.pallas_call(
        _proj_body,
        grid=grid,
        in_specs=[
            pl.BlockSpec((B, D), lambda i: (i, 0)),
            pl.BlockSpec((D, H * C), lambda i: (0, 0)),
            pl.BlockSpec((H, C), lambda i: (0, 0)),
            pl.BlockSpec((H, C), lambda i: (0, 0)),
        ],
        out_specs=[
            pl.BlockSpec((H, B, C), lambda i: (0, i, 0)),
            pl.BlockSpec((B, H), lambda i: (i, 0)),
            pl.BlockSpec((B, H), lambda i: (i, 0)),
        ],
        out_shape=[
            jax.ShapeDtypeStruct((H, N, C), jnp.float32),
            jax.ShapeDtypeStruct((N, H), jnp.float32),
            jax.ShapeDtypeStruct((N, H), jnp.float32),
        ],
    )(x, W, att_src, att_dst)


def _edge_body(src_hbm, dst_hbm, asrc_hbm, adst_hbm, h_hbm, agg_hbm,
               src_v, dst_v, rows_v, asg_v, adg_v, sxp_v, den_v,
               coef_v, zflat_v, sem_r, sem_g, sem_d, sem_s, den_sh, out_sh):
    c = lax.axis_index("c")
    s = lax.axis_index("s")
    base = s * EPT
    iota = lax.iota(jnp.int32, 16)
    zf16 = jnp.zeros((16,), jnp.float32)

    pltpu.sync_copy(src_hbm.at[pl.ds(base, EPT)], src_v)
    pltpu.sync_copy(dst_hbm.at[pl.ds(base, EPT)], dst_v)

    def _zero_zflat(i, _):
        zflat_v[pl.ds(i * 16, 16)] = zf16
        return 0
    lax.fori_loop(0, ZFW // 16, _zero_zflat, 0)

    def _zero_rows(i, _):
        rows_v[i // 8, pl.ds((i % 8) * 16, 16)] = zf16
        return 0
    lax.fori_loop(0, BPB * 8, _zero_rows, 0)

    def _zero_out(i, _):
        pltpu.sync_copy(rows_v, out_sh.at[pl.ds(s * ROWB + i * BPB, BPB)])
        return 0
    lax.fori_loop(0, ROWB // BPB, _zero_out, 0)

    @pl.when(s == 0)
    def _():
        def _zero_den(i, _):
            pltpu.sync_copy(zflat_v, den_sh.at[pl.ds(i * ZFW, ZFW)])
            return 0
        lax.fori_loop(0, NP // ZFW, _zero_den, 0)

    plsc.subcore_barrier()

    def _gather_logits(b):
        ds_ = []
        for g in range(GPB):
            sv = src_v[pl.ds(b * BPB + g * 16, 16)]
            dv = dst_v[pl.ds(b * BPB + g * 16, 16)]
            ds_.append(pltpu.async_copy(
                asrc_hbm.at[sv * H + c],
                asg_v.at[pl.ds(g * 16, 16)], sem_g))
            ds_.append(pltpu.async_copy(
                adst_hbm.at[dv * H + c],
                adg_v.at[pl.ds(g * 16, 16)], sem_g))
        return ds_

    def _sxp_group(b, g):
        al = asg_v[pl.ds(g * 16, 16)] + adg_v[pl.ds(g * 16, 16)]
        al = jnp.where(al >= 0.0, al, 0.2 * al)
        sxp = jnp.exp(al)
        valid = (base + b * BPB + g * 16 + iota) < ETOT
        return jnp.where(valid, sxp, 0.0)

    def _p1(b, _):
        ga = _gather_logits(b)
        for d in ga:
            d.wait()
        for g in range(GPB):
            sxp_v[pl.ds(g * 16, 16)] = _sxp_group(b, g)
        sc_ = []
        for g in range(GPB):
            dv = dst_v[pl.ds(b * BPB + g * 16, 16)]
            sc_.append(pltpu.async_copy(
                sxp_v.at[pl.ds(g * 16, 16)], den_sh.at[dv], sem_s, add=True))
        for d in sc_:
            d.wait()
        return 0
    lax.fori_loop(0, NBAT, _p1, 0)

    plsc.subcore_barrier()

    def _p2(b, _):
        dr = pltpu.async_copy(h_hbm.at[c].at[src_v.at[pl.ds(b * BPB, BPB)]],
                              rows_v, sem_r)
        dd = pltpu.async_copy(den_sh.at[dst_v.at[pl.ds(b * BPB, BPB)]],
                              den_v, sem_d)
        ga = _gather_logits(b)
        dr.wait()
        dd.wait()
        for d in ga:
            d.wait()
        for g in range(GPB):
            sxp = _sxp_group(b, g)
            coef_v[pl.ds(16, 16)] = sxp / (den_v[pl.ds(g * 16, 16)] + 1e-16)
            for r in range(16):
                cv = plsc.load_gather(coef_v,
                                      [jnp.full((16,), 16 + r, jnp.int32)])
                for q in range(8):
                    rows_v[g * 16 + r, pl.ds(q * 16, 16)] = (
                        rows_v[g * 16 + r, pl.ds(q * 16, 16)] * cv)
        sc_ = []
        for g in range(GPB):
            dv = dst_v[pl.ds(b * BPB + g * 16, 16)]
            sc_.append(pltpu.async_copy(
                rows_v.at[pl.ds(g * 16, 16), :], out_sh.at[dv], sem_s,
                add=True))
        for d in sc_:
            d.wait()
        return 0
    lax.fori_loop(0, NBAT, _p2, 0)

    plsc.subcore_barrier()
    pltpu.sync_copy(out_sh.at[pl.ds(s * ROWB, ROWB)],
                    agg_hbm.at[c, pl.ds(s * ROWB, ROWB), :])


def _edge_aggregate(src, dst, asrc_flat, adst_flat, h_flat):
    mesh = plsc.VectorSubcoreMesh(core_axis_name="c", subcore_axis_name="s")
    f = pl.kernel(
        _edge_body,
        out_type=jax.ShapeDtypeStruct((H, NP, C), jnp.float32),
        mesh=mesh,
        compiler_params=pltpu.CompilerParams(use_tc_tiling_on_sc=False,
                                             needs_layout_passes=False),
        scratch_types=[
            pltpu.VMEM((EPT,), jnp.int32),
            pltpu.VMEM((EPT,), jnp.int32),
            pltpu.VMEM((BPB, C), jnp.float32),
            pltpu.VMEM((BPB,), jnp.float32),
            pltpu.VMEM((BPB,), jnp.float32),
            pltpu.VMEM((BPB,), jnp.float32),
            pltpu.VMEM((BPB,), jnp.float32),
            pltpu.VMEM((32,), jnp.float32),
            pltpu.VMEM((ZFW,), jnp.float32),
            pltpu.SemaphoreType.DMA,
            pltpu.SemaphoreType.DMA,
            pltpu.SemaphoreType.DMA,
            pltpu.SemaphoreType.DMA,
            pltpu.VMEM_SHARED((NP,), jnp.float32),
            pltpu.VMEM_SHARED((NP, C), jnp.float32),
        ],
    )
    return f(src, dst, asrc_flat, adst_flat, h_flat)


def _mlp_body(agg_ref, bias_ref, w1_ref, b1_ref, w2_ref, b2_ref, z_ref):
    feat = jnp.concatenate([agg_ref[0], agg_ref[1]], axis=1)
    out = feat + bias_ref[...]
    out = jnp.where(out >= 0.0, out, 0.01 * out)
    z1 = jnp.dot(out, w1_ref[...], preferred_element_type=jnp.float32) + b1_ref[...]
    z1 = jnp.where(z1 >= 0.0, z1, 0.01 * z1)
    z_ref[...] = jnp.dot(z1, w2_ref[...], preferred_element_type=jnp.float32) + b2_ref[...]


def _mlp(agg, bias, W1, b1, W2, b2):
    B = 1000
    return pl.pallas_call(
        _mlp_body,
        grid=(N // B,),
        in_specs=[
            pl.BlockSpec((H, B, C), lambda i: (0, i, 0)),
            pl.BlockSpec((1, H * C), lambda i: (0, 0)),
            pl.BlockSpec((H * C, 64), lambda i: (0, 0)),
            pl.BlockSpec((1, 64), lambda i: (0, 0)),
            pl.BlockSpec((64, 3), lambda i: (0, 0)),
            pl.BlockSpec((1, 3), lambda i: (0, 0)),
        ],
        out_specs=pl.BlockSpec((B, 3), lambda i: (i, 0)),
        out_shape=jax.ShapeDtypeStruct((N, 3), jnp.float32),
    )(agg, bias, W1, b1, W2, b2)


def _cdist_body(zi_ref, zjt_ref, d_ref):
    zi = zi_ref[...]
    zjt = zjt_ref[...]
    g = jnp.dot(zi, zjt, preferred_element_type=jnp.float32)
    sqi = jnp.sum(zi * zi, axis=1, keepdims=True)
    sqj = jnp.sum(zjt * zjt, axis=0, keepdims=True)
    d2 = sqi + sqj - 2.0 * g
    d2 = jnp.maximum(d2, 0.0)
    d = jnp.sqrt(jnp.where(d2 > 0.0, d2, 1.0))
    d_ref[...] = jnp.where(d2 > 0.0, d, 0.0)


def _cdist(z, zT):
    B = 1024
    g = (N + B - 1) // B
    return pl.pallas_call(
        _cdist_body,
        grid=(g, g),
        in_specs=[
            pl.BlockSpec((B, 3), lambda i, j: (i, 0)),
            pl.BlockSpec((3, B), lambda i, j: (0, j)),
        ],
        out_specs=pl.BlockSpec((B, B), lambda i, j: (i, j)),
        out_shape=jax.ShapeDtypeStruct((N, N), jnp.float32),
    )(z, zT)


def kernel(x, edge_index, W, att_src, att_dst, bias, W1, b1, W2, b2):
    loop = jnp.arange(N, dtype=edge_index.dtype)
    pad = jnp.zeros((EPAD - ETOT,), dtype=edge_index.dtype)
    src = jnp.concatenate([edge_index[0], loop, pad])
    dst = jnp.concatenate([edge_index[1], loop, pad])

    h, a_src, a_dst = _project(x, W, att_src.reshape(H, C), att_dst.reshape(H, C))
    agg = _edge_aggregate(src, dst, a_src.reshape(-1), a_dst.reshape(-1), h)
    z = _mlp(agg, bias.reshape(1, H * C), W1, b1.reshape(1, 64), W2,
             b2.reshape(1, 3))
    return _cdist(z, z.T)
